# baseline (speedup 1.0000x reference)
import functools

import numpy as np

import jax
import jax.numpy as jnp
from jax import lax
from jax.experimental import pallas as pl
from jax.experimental.pallas import tpu as pltpu
from jax.experimental.pallas import tpu_sc as plsc

_BS, _S, _P, _R, _D = 1024, 50, 100000, 1000, 64
_K = 10

_NC, _NS = 2, 16
_NW = _NC * _NS
_NSEQ = _BS * _S
_CHUNK = _NSEQ // _NW

_TP = 1024
_NB = -(-_P // _TP)
_PPAD = _NB * _TP

_Z = np.int32(0)

_UB = 128
_SPAD = 56

_GCH = 100
_GROWS = _CHUNK // _GCH


def _sc_seq_gather(seq3d, rid_table):
    mesh = plsc.VectorSubcoreMesh(core_axis_name="c", subcore_axis_name="s")

    @functools.partial(
        pl.kernel,
        mesh=mesh,
        out_type=jax.ShapeDtypeStruct((_NW, _GROWS, _GCH), jnp.int32),
        scratch_types=[
            pltpu.VMEM((_GROWS, _GCH), jnp.int32),
            pltpu.VMEM((_GROWS, _GCH), jnp.int32),
            pltpu.SemaphoreType.DMA,
        ],
    )
    def k(seq_hbm, table_hbm, out_hbm, idx_v, out_v, sem):
        wid = (lax.axis_index("s") * jnp.int32(_NC)
               + lax.axis_index("c")).astype(jnp.int32)
        pltpu.sync_copy(seq_hbm.at[wid], idx_v)
        copies = [
            pltpu.async_copy(table_hbm.at[idx_v.at[jnp.int32(j)]],
                             out_v.at[jnp.int32(j)], sem)
            for j in range(_GROWS)
        ]
        for c in copies:
            c.wait()
        pltpu.sync_copy(out_v, out_hbm.at[wid])

    return k(seq3d, rid_table)


def _pref_body(rid_ref, len_ref, rw_ref, alpha_ref, out_ref, acc_ref):
    L = len_ref[...]
    start = jnp.maximum(L - _K, 0)
    cnt = jnp.maximum(L - start, 1).astype(jnp.float32)
    w0 = alpha_ref[...] / cnt
    iota_r = lax.broadcasted_iota(jnp.int32, (1, _R), 1)
    iota_s = lax.broadcasted_iota(jnp.int32, (1, _SPAD), 1)
    rid = rid_ref[...]
    acc_ref[...] = jnp.zeros((_UB, _R), jnp.float32)

    def step(s, c):
        rid_s = jnp.sum(jnp.where(iota_s == s, rid, jnp.int32(0)),
                        axis=1, keepdims=True,
                        dtype=jnp.int32)
        valid = (s >= start) & (s < L)
        w = jnp.where(valid, w0, 0.0)
        oh = (rid_s == iota_r).astype(jnp.float32)
        acc_ref[...] += w * oh
        return c

    lax.fori_loop(jnp.int32(0), jnp.int32(_S), step, jnp.int32(0))
    u = jnp.dot(acc_ref[...], rw_ref[...],
                preferred_element_type=jnp.float32)
    out_ref[...] = lax.dot_general(u, rw_ref[...],
                                   (((1,), (1,)), ((), ())),
                                   preferred_element_type=jnp.float32)


def _pref_call(rid_pad, len2d, region_weight, alpha2d):
    return pl.pallas_call(
        _pref_body,
        grid=(_BS // _UB,),
        in_specs=[
            pl.BlockSpec((_UB, _SPAD), lambda i: (i, _Z)),
            pl.BlockSpec((_UB, 1), lambda i: (i, _Z)),
            pl.BlockSpec((_R, _D), lambda i: (_Z, _Z)),
            pl.BlockSpec((1, 1), lambda i: (_Z, _Z)),
        ],
        out_specs=pl.BlockSpec((_UB, _R), lambda i: (i, _Z)),
        out_shape=jax.ShapeDtypeStruct((_BS, _R), jnp.float32),
        scratch_shapes=[pltpu.VMEM((_UB, _R), jnp.float32)],
    )(rid_pad, len2d, region_weight, alpha2d)


_RPW = _BS // _NW
_CC = 2000
_NCH = _P // _CC
_RING = 4


def _sc_main(pred_base, rid32, ru):
    mesh = plsc.VectorSubcoreMesh(core_axis_name="c", subcore_axis_name="s")

    @functools.partial(
        pl.kernel,
        mesh=mesh,
        out_type=jax.ShapeDtypeStruct((_BS, _P), jnp.float32),
        scratch_types=[
            pltpu.VMEM((_RPW, _R), jnp.float32),
            pltpu.VMEM((_CC,), jnp.int32),
            pltpu.VMEM((2, _RPW // 2, _CC), jnp.float32),
            pltpu.SemaphoreType.DMA,
            pltpu.SemaphoreType.DMA,
        ],
        compiler_params=pltpu.CompilerParams(use_tc_tiling_on_sc=False,
                                             needs_layout_passes=False),
    )
    def k(pred_hbm, rid_hbm, ru_hbm, out_hbm, tab_v, idx_v, ring, isem, osem):
        wid = (lax.axis_index("s") * jnp.int32(_NC)
               + lax.axis_index("c")).astype(jnp.int32)
        row0 = wid * jnp.int32(_RPW)
        hrows = _RPW // 2
        pltpu.sync_copy(ru_hbm.at[pl.ds(row0, _RPW)], tab_v)

        def chunk_body(c, carry):
            col0 = c * jnp.int32(_CC)
            pltpu.sync_copy(rid_hbm.at[pl.ds(col0, _CC)], idx_v)
            ins = []
            for h in range(2):
                ins.append(pltpu.async_copy(
                    pred_hbm.at[pl.ds(row0 + np.int32(h * hrows), hrows),
                                pl.ds(col0, _CC)],
                    ring.at[np.int32(h)], isem))
            outs = []
            for h in range(2):
                ins[h].wait()
                half = ring.at[np.int32(h)]

                def slice_body(i, cc, _half=half, _h=h):
                    off = i * jnp.int32(16)
                    idx = idx_v[pl.ds(off, 16)]
                    for rr in range(hrows):
                        rowvec = jnp.full((16,), jnp.int32(_h * hrows + rr),
                                          jnp.int32)
                        vals = plsc.load_gather(tab_v, [rowvec, idx])
                        plsc.addupdate(
                            _half.at[np.int32(rr)].at[pl.ds(off, 16)], vals)
                    return cc

                lax.fori_loop(jnp.int32(0), jnp.int32(_CC // 16),
                              slice_body, jnp.int32(0))
                outs.append(pltpu.async_copy(
                    half,
                    out_hbm.at[pl.ds(row0 + np.int32(h * hrows), hrows),
                               pl.ds(col0, _CC)],
                    osem))
            for o in outs:
                o.wait()
            return carry

        lax.fori_loop(jnp.int32(0), jnp.int32(_NCH), chunk_body, jnp.int32(0))

    return k(pred_base, rid32, ru)


def _main_body(rid_ref, pred_ref, up_ref, rwt_ref, out_ref):
    rid_row = rid_ref[0]
    iota_r = lax.broadcasted_iota(jnp.int32, (_R, 1), 0)
    oh_t = (rid_row == iota_r).astype(jnp.bfloat16)
    e_t = jnp.dot(rwt_ref[...].astype(jnp.bfloat16), oh_t,
                  preferred_element_type=jnp.float32)
    score = jnp.dot(up_ref[...].astype(jnp.bfloat16),
                    e_t.astype(jnp.bfloat16),
                    preferred_element_type=jnp.float32)
    out_ref[...] = pred_ref[...] + score


def _main_call(rid3d, pred_base, up, rwt):
    return pl.pallas_call(
        _main_body,
        grid=(_NB,),
        in_specs=[
            pl.BlockSpec((1, 1, _TP), lambda i: (i, _Z, _Z)),
            pl.BlockSpec((_BS, _TP), lambda i: (_Z, i)),
            pl.BlockSpec((_BS, _D), lambda i: (_Z, _Z)),
            pl.BlockSpec((_D, _R), lambda i: (_Z, _Z)),
        ],
        out_specs=pl.BlockSpec((_BS, _TP), lambda i: (_Z, i)),
        out_shape=jax.ShapeDtypeStruct((_BS, _P), jnp.float32),
        compiler_params=pltpu.CompilerParams(
            dimension_semantics=("arbitrary",)),
    )(rid3d, pred_base, up, rwt)


def kernel(pred_base, user_seq, user_seq_len, poi_region_id, region_weight, alpha):
    seq3d = user_seq.astype(jnp.int32).reshape(_NW, _GROWS, _GCH)
    rid32 = poi_region_id.astype(jnp.int32)
    rid_seq = _sc_seq_gather(seq3d, rid32)
    rid_pad = jnp.pad(rid_seq.reshape(_BS, _S), ((0, 0), (0, _SPAD - _S)))
    len2d = user_seq_len.astype(jnp.int32).reshape(_BS, 1)
    alpha2d = alpha.astype(jnp.float32).reshape(1, 1)
    ru = _pref_call(rid_pad, len2d, region_weight, alpha2d)
    return _sc_main(pred_base, rid32, ru)


# speedup vs baseline: 1.2435x; 1.2435x over previous
import functools

import numpy as np

import jax
import jax.numpy as jnp
from jax import lax
from jax.experimental import pallas as pl
from jax.experimental.pallas import tpu as pltpu
from jax.experimental.pallas import tpu_sc as plsc

_BS, _S, _P, _R, _D = 1024, 50, 100000, 1000, 64
_K = 10

_NC, _NS = 2, 16
_NW = _NC * _NS
_NSEQ = _BS * _S
_CHUNK = _NSEQ // _NW

_TP = 1024
_NB = -(-_P // _TP)
_PPAD = _NB * _TP

_Z = np.int32(0)

_UB = 128
_SPAD = 56

_GCH = 100
_GROWS = _CHUNK // _GCH


def _sc_seq_gather(seq3d, rid_table):
    mesh = plsc.VectorSubcoreMesh(core_axis_name="c", subcore_axis_name="s")

    @functools.partial(
        pl.kernel,
        mesh=mesh,
        out_type=jax.ShapeDtypeStruct((_NW, _GROWS, _GCH), jnp.int32),
        scratch_types=[
            pltpu.VMEM((_GROWS, _GCH), jnp.int32),
            pltpu.VMEM((_GROWS, _GCH), jnp.int32),
            pltpu.SemaphoreType.DMA,
        ],
    )
    def k(seq_hbm, table_hbm, out_hbm, idx_v, out_v, sem):
        wid = (lax.axis_index("s") * jnp.int32(_NC)
               + lax.axis_index("c")).astype(jnp.int32)
        pltpu.sync_copy(seq_hbm.at[wid], idx_v)
        copies = [
            pltpu.async_copy(table_hbm.at[idx_v.at[jnp.int32(j)]],
                             out_v.at[jnp.int32(j)], sem)
            for j in range(_GROWS)
        ]
        for c in copies:
            c.wait()
        pltpu.sync_copy(out_v, out_hbm.at[wid])

    return k(seq3d, rid_table)


def _pref_body(rid_ref, len_ref, rw_ref, alpha_ref, out_ref, acc_ref):
    L = len_ref[...]
    start = jnp.maximum(L - _K, 0)
    cnt = jnp.maximum(L - start, 1).astype(jnp.float32)
    w0 = alpha_ref[...] / cnt
    iota_r = lax.broadcasted_iota(jnp.int32, (1, _R), 1)
    iota_s = lax.broadcasted_iota(jnp.int32, (1, _SPAD), 1)
    rid = rid_ref[...]
    acc_ref[...] = jnp.zeros((_UB, _R), jnp.float32)

    def step(s, c):
        rid_s = jnp.sum(jnp.where(iota_s == s, rid, jnp.int32(0)),
                        axis=1, keepdims=True,
                        dtype=jnp.int32)
        valid = (s >= start) & (s < L)
        w = jnp.where(valid, w0, 0.0)
        oh = (rid_s == iota_r).astype(jnp.float32)
        acc_ref[...] += w * oh
        return c

    lax.fori_loop(jnp.int32(0), jnp.int32(_S), step, jnp.int32(0))
    u = jnp.dot(acc_ref[...], rw_ref[...],
                preferred_element_type=jnp.float32)
    out_ref[...] = lax.dot_general(u, rw_ref[...],
                                   (((1,), (1,)), ((), ())),
                                   preferred_element_type=jnp.float32)


def _pref_call(rid_pad, len2d, region_weight, alpha2d):
    return pl.pallas_call(
        _pref_body,
        grid=(_BS // _UB,),
        in_specs=[
            pl.BlockSpec((_UB, _SPAD), lambda i: (i, _Z)),
            pl.BlockSpec((_UB, 1), lambda i: (i, _Z)),
            pl.BlockSpec((_R, _D), lambda i: (_Z, _Z)),
            pl.BlockSpec((1, 1), lambda i: (_Z, _Z)),
        ],
        out_specs=pl.BlockSpec((_UB, _R), lambda i: (i, _Z)),
        out_shape=jax.ShapeDtypeStruct((_BS, _R), jnp.float32),
        scratch_shapes=[pltpu.VMEM((_UB, _R), jnp.float32)],
    )(rid_pad, len2d, region_weight, alpha2d)


_RPW = _BS // _NW
_CC = 2000
_NCH = _P // _CC
_RING = 4


def _sc_main(pred_base, rid32, ru):
    mesh = plsc.VectorSubcoreMesh(core_axis_name="c", subcore_axis_name="s")

    @functools.partial(
        pl.kernel,
        mesh=mesh,
        out_type=jax.ShapeDtypeStruct((_BS, _P), jnp.float32),
        scratch_types=[
            pltpu.VMEM((_RPW, _R), jnp.float32),
            pltpu.VMEM((_CC,), jnp.int32),
            pltpu.VMEM((2, _RPW // 2, _CC), jnp.float32),
            pltpu.SemaphoreType.DMA,
            pltpu.SemaphoreType.DMA,
        ],
        compiler_params=pltpu.CompilerParams(use_tc_tiling_on_sc=False,
                                             needs_layout_passes=False),
    )
    def k(pred_hbm, rid_hbm, ru_hbm, out_hbm, tab_v, idx_v, ring, isem, osem):
        wid = (lax.axis_index("s") * jnp.int32(_NC)
               + lax.axis_index("c")).astype(jnp.int32)
        row0 = wid * jnp.int32(_RPW)
        hrows = _RPW // 2
        pltpu.sync_copy(ru_hbm.at[pl.ds(row0, _RPW)], tab_v)

        def chunk_body(c, carry):
            col0 = c * jnp.int32(_CC)
            pltpu.sync_copy(rid_hbm.at[pl.ds(col0, _CC)], idx_v)
            ins = []
            for h in range(2):
                ins.append(pltpu.async_copy(
                    pred_hbm.at[pl.ds(row0 + np.int32(h * hrows), hrows),
                                pl.ds(col0, _CC)],
                    ring.at[np.int32(h)], isem))
            outs = []
            for h in range(2):
                ins[h].wait()
                half = ring.at[np.int32(h)]

                @plsc.parallel_loop(np.int32(0), np.int32(_CC),
                                    step=np.int32(16), unroll=4)
                def slice_body(off, _half=half, _h=h):
                    idx = idx_v[pl.ds(off, 16)]
                    for rr in range(hrows):
                        rowvec = jnp.full((16,), jnp.int32(_h * hrows + rr),
                                          jnp.int32)
                        vals = plsc.load_gather(tab_v, [rowvec, idx])
                        plsc.addupdate(
                            _half.at[np.int32(rr)].at[pl.ds(off, 16)], vals)
                outs.append(pltpu.async_copy(
                    half,
                    out_hbm.at[pl.ds(row0 + np.int32(h * hrows), hrows),
                               pl.ds(col0, _CC)],
                    osem))
            for o in outs:
                o.wait()
            return carry

        lax.fori_loop(jnp.int32(0), jnp.int32(_NCH), chunk_body, jnp.int32(0))

    return k(pred_base, rid32, ru)


def _main_body(rid_ref, pred_ref, up_ref, rwt_ref, out_ref):
    rid_row = rid_ref[0]
    iota_r = lax.broadcasted_iota(jnp.int32, (_R, 1), 0)
    oh_t = (rid_row == iota_r).astype(jnp.bfloat16)
    e_t = jnp.dot(rwt_ref[...].astype(jnp.bfloat16), oh_t,
                  preferred_element_type=jnp.float32)
    score = jnp.dot(up_ref[...].astype(jnp.bfloat16),
                    e_t.astype(jnp.bfloat16),
                    preferred_element_type=jnp.float32)
    out_ref[...] = pred_ref[...] + score


def _main_call(rid3d, pred_base, up, rwt):
    return pl.pallas_call(
        _main_body,
        grid=(_NB,),
        in_specs=[
            pl.BlockSpec((1, 1, _TP), lambda i: (i, _Z, _Z)),
            pl.BlockSpec((_BS, _TP), lambda i: (_Z, i)),
            pl.BlockSpec((_BS, _D), lambda i: (_Z, _Z)),
            pl.BlockSpec((_D, _R), lambda i: (_Z, _Z)),
        ],
        out_specs=pl.BlockSpec((_BS, _TP), lambda i: (_Z, i)),
        out_shape=jax.ShapeDtypeStruct((_BS, _P), jnp.float32),
        compiler_params=pltpu.CompilerParams(
            dimension_semantics=("arbitrary",)),
    )(rid3d, pred_base, up, rwt)


def kernel(pred_base, user_seq, user_seq_len, poi_region_id, region_weight, alpha):
    seq3d = user_seq.astype(jnp.int32).reshape(_NW, _GROWS, _GCH)
    rid32 = poi_region_id.astype(jnp.int32)
    rid_seq = _sc_seq_gather(seq3d, rid32)
    rid_pad = jnp.pad(rid_seq.reshape(_BS, _S), ((0, 0), (0, _SPAD - _S)))
    len2d = user_seq_len.astype(jnp.int32).reshape(_BS, 1)
    alpha2d = alpha.astype(jnp.float32).reshape(1, 1)
    ru = _pref_call(rid_pad, len2d, region_weight, alpha2d)
    return _sc_main(pred_base, rid32, ru)


# speedup vs baseline: 2.7017x; 2.1727x over previous
import functools

import numpy as np

import jax
import jax.numpy as jnp
from jax import lax
from jax.experimental import pallas as pl
from jax.experimental.pallas import tpu as pltpu
from jax.experimental.pallas import tpu_sc as plsc

_BS, _S, _P, _R, _D = 1024, 50, 100000, 1000, 64
_K = 10

_NC, _NS = 2, 16
_NW = _NC * _NS
_NSEQ = _BS * _S
_CHUNK = _NSEQ // _NW

_TP = 1024
_NB = -(-_P // _TP)
_PPAD = _NB * _TP

_Z = np.int32(0)

_UB = 128
_SPAD = 56

_GCH = 100
_GROWS = _CHUNK // _GCH


def _sc_seq_gather(seq3d, rid_table):
    mesh = plsc.VectorSubcoreMesh(core_axis_name="c", subcore_axis_name="s")

    @functools.partial(
        pl.kernel,
        mesh=mesh,
        out_type=jax.ShapeDtypeStruct((_NW, _GROWS, _GCH), jnp.int32),
        scratch_types=[
            pltpu.VMEM((_GROWS, _GCH), jnp.int32),
            pltpu.VMEM((_GROWS, _GCH), jnp.int32),
            pltpu.SemaphoreType.DMA,
        ],
    )
    def k(seq_hbm, table_hbm, out_hbm, idx_v, out_v, sem):
        wid = (lax.axis_index("s") * jnp.int32(_NC)
               + lax.axis_index("c")).astype(jnp.int32)
        pltpu.sync_copy(seq_hbm.at[wid], idx_v)
        copies = [
            pltpu.async_copy(table_hbm.at[idx_v.at[jnp.int32(j)]],
                             out_v.at[jnp.int32(j)], sem)
            for j in range(_GROWS)
        ]
        for c in copies:
            c.wait()
        pltpu.sync_copy(out_v, out_hbm.at[wid])

    return k(seq3d, rid_table)


def _pref_body(rid_ref, len_ref, rw_ref, alpha_ref, out_ref, acc_ref):
    L = len_ref[...]
    start = jnp.maximum(L - _K, 0)
    cnt = jnp.maximum(L - start, 1).astype(jnp.float32)
    w0 = alpha_ref[...] / cnt
    iota_r = lax.broadcasted_iota(jnp.int32, (1, _R), 1)
    iota_s = lax.broadcasted_iota(jnp.int32, (1, _SPAD), 1)
    rid = rid_ref[...]
    acc_ref[...] = jnp.zeros((_UB, _R), jnp.float32)

    def step(s, c):
        rid_s = jnp.sum(jnp.where(iota_s == s, rid, jnp.int32(0)),
                        axis=1, keepdims=True,
                        dtype=jnp.int32)
        valid = (s >= start) & (s < L)
        w = jnp.where(valid, w0, 0.0)
        oh = (rid_s == iota_r).astype(jnp.float32)
        acc_ref[...] += w * oh
        return c

    lax.fori_loop(jnp.int32(0), jnp.int32(_S), step, jnp.int32(0))
    out_ref[...] = jnp.dot(acc_ref[...], rw_ref[...],
                           preferred_element_type=jnp.float32)


def _pref_call(rid_pad, len2d, region_weight, alpha2d):
    return pl.pallas_call(
        _pref_body,
        grid=(_BS // _UB,),
        in_specs=[
            pl.BlockSpec((_UB, _SPAD), lambda i: (i, _Z)),
            pl.BlockSpec((_UB, 1), lambda i: (i, _Z)),
            pl.BlockSpec((_R, _D), lambda i: (_Z, _Z)),
            pl.BlockSpec((1, 1), lambda i: (_Z, _Z)),
        ],
        out_specs=pl.BlockSpec((_UB, _D), lambda i: (i, _Z)),
        out_shape=jax.ShapeDtypeStruct((_BS, _D), jnp.float32),
        scratch_shapes=[pltpu.VMEM((_UB, _R), jnp.float32)],
    )(rid_pad, len2d, region_weight, alpha2d)


def _main_body(rid_ref, pred_ref, up_ref, rwt_ref, out_ref):
    rid_row = rid_ref[0]
    iota_r = lax.broadcasted_iota(jnp.int32, (_R, 1), 0)
    oh_t = (rid_row == iota_r).astype(jnp.bfloat16)
    e_t = jnp.dot(rwt_ref[...].astype(jnp.bfloat16), oh_t,
                  preferred_element_type=jnp.float32)
    score = jnp.dot(up_ref[...].astype(jnp.bfloat16),
                    e_t.astype(jnp.bfloat16),
                    preferred_element_type=jnp.float32)
    out_ref[...] = pred_ref[...] + score


def _main_call(rid3d, pred_base, up, rwt):
    return pl.pallas_call(
        _main_body,
        grid=(_NB,),
        in_specs=[
            pl.BlockSpec((1, 1, _TP), lambda i: (i, _Z, _Z)),
            pl.BlockSpec((_BS, _TP), lambda i: (_Z, i)),
            pl.BlockSpec((_BS, _D), lambda i: (_Z, _Z)),
            pl.BlockSpec((_D, _R), lambda i: (_Z, _Z)),
        ],
        out_specs=pl.BlockSpec((_BS, _TP), lambda i: (_Z, i)),
        out_shape=jax.ShapeDtypeStruct((_BS, _P), jnp.float32),
        compiler_params=pltpu.CompilerParams(
            dimension_semantics=("arbitrary",)),
    )(rid3d, pred_base, up, rwt)


def kernel(pred_base, user_seq, user_seq_len, poi_region_id, region_weight, alpha):
    seq3d = user_seq.astype(jnp.int32).reshape(_NW, _GROWS, _GCH)
    rid32 = poi_region_id.astype(jnp.int32)
    rid_seq = _sc_seq_gather(seq3d, rid32)
    rid_pad = jnp.pad(rid_seq.reshape(_BS, _S), ((0, 0), (0, _SPAD - _S)))
    len2d = user_seq_len.astype(jnp.int32).reshape(_BS, 1)
    alpha2d = alpha.astype(jnp.float32).reshape(1, 1)
    up = _pref_call(rid_pad, len2d, region_weight, alpha2d)
    rid3d = jnp.pad(rid32, (0, _PPAD - _P)).reshape(_NB, 1, _TP)
    return _main_call(rid3d, pred_base, up, region_weight.T)
